# baseline (speedup 1.0000x reference)
import functools

import jax
import jax.numpy as jnp
from jax import lax
from jax.experimental import pallas as pl
from jax.experimental.pallas import tpu as pltpu
from jax.experimental.pallas import tpu_sc as plsc

EMBED_DIM = 64
NUM_CORES = 2
NUM_SUBCORES = 16
NUM_WORKERS = NUM_CORES * NUM_SUBCORES
CHUNK = 800


@functools.partial(jax.jit, static_argnames=("n_chunks",))
def _sc_gather(ids_flat, table, n_chunks):
    bpw = ids_flat.shape[0] // NUM_WORKERS
    out_type = jax.ShapeDtypeStruct((ids_flat.shape[0], EMBED_DIM), table.dtype)
    mesh = plsc.VectorSubcoreMesh(core_axis_name="c", subcore_axis_name="s")

    @functools.partial(
        pl.kernel,
        mesh=mesh,
        out_type=out_type,
        scratch_types=[
            pltpu.VMEM((CHUNK,), jnp.int32),
            pltpu.VMEM((CHUNK, EMBED_DIM), jnp.float32),
            pltpu.SemaphoreType.DMA,
        ],
        compiler_params=pltpu.CompilerParams(use_tc_tiling_on_sc=False),
    )
    def gather_kernel(ids_hbm, table_hbm, out_hbm, idx_v, rows_v, sem):
        wid = lax.axis_index("s") * NUM_CORES + lax.axis_index("c")
        base = wid * bpw

        def body(j, carry):
            off = base + j * CHUNK
            pltpu.sync_copy(ids_hbm.at[pl.ds(off, CHUNK)], idx_v)
            pltpu.async_copy(table_hbm.at[idx_v], rows_v, sem).wait()
            pltpu.sync_copy(rows_v, out_hbm.at[pl.ds(off, CHUNK)])
            return carry

        lax.fori_loop(0, n_chunks, body, 0)

    return gather_kernel(ids_flat, table)


def kernel(input_ids, table):
    batch, seq = input_ids.shape
    total = batch * seq
    ids_flat = input_ids.reshape(total).astype(jnp.int32)
    assert total % (NUM_WORKERS * CHUNK) == 0
    n_chunks = total // (NUM_WORKERS * CHUNK)
    out = _sc_gather(ids_flat, table, n_chunks)
    return out.reshape(batch, seq, EMBED_DIM)


# speedup vs baseline: 1.0052x; 1.0052x over previous
import functools

import jax
import jax.numpy as jnp
from jax import lax
from jax.experimental import pallas as pl
from jax.experimental.pallas import tpu as pltpu
from jax.experimental.pallas import tpu_sc as plsc

EMBED_DIM = 64
NUM_CORES = 2
NUM_SUBCORES = 16
NUM_WORKERS = NUM_CORES * NUM_SUBCORES
CHUNK = 800


@functools.partial(jax.jit, static_argnames=("n_chunks",))
def _sc_gather(ids_flat, table, n_chunks):
    bpw = ids_flat.shape[0] // NUM_WORKERS
    out_type = jax.ShapeDtypeStruct((ids_flat.shape[0], EMBED_DIM), table.dtype)
    mesh = plsc.VectorSubcoreMesh(core_axis_name="c", subcore_axis_name="s")

    @functools.partial(
        pl.kernel,
        mesh=mesh,
        out_type=out_type,
        scratch_types=[
            pltpu.VMEM((bpw,), jnp.int32),
            pltpu.VMEM((CHUNK, EMBED_DIM), jnp.float32),
            pltpu.VMEM((CHUNK, EMBED_DIM), jnp.float32),
            pltpu.SemaphoreType.DMA,
            pltpu.SemaphoreType.DMA,
            pltpu.SemaphoreType.DMA,
            pltpu.SemaphoreType.DMA,
        ],
        compiler_params=pltpu.CompilerParams(use_tc_tiling_on_sc=False),
    )
    def gather_kernel(ids_hbm, table_hbm, out_hbm, idx_v, rows0, rows1,
                      sg0, sg1, sw0, sw1):
        wid = lax.axis_index("s") * NUM_CORES + lax.axis_index("c")
        base = wid * bpw
        rows = (rows0, rows1)
        sg = (sg0, sg1)
        sw = (sw0, sw1)

        pltpu.sync_copy(ids_hbm.at[pl.ds(base, bpw)], idx_v)

        def gcopy(j):
            return pltpu.make_async_copy(
                table_hbm.at[idx_v.at[pl.ds(j * CHUNK, CHUNK)]],
                rows[j % 2], sg[j % 2])

        def wcopy(j):
            return pltpu.make_async_copy(
                rows[j % 2], out_hbm.at[pl.ds(base + j * CHUNK, CHUNK)],
                sw[j % 2])

        gcopy(0).start()
        for j in range(n_chunks):
            if j >= 1:
                wcopy(j - 1).wait()
            gcopy(j).wait()
            if j + 1 < n_chunks:
                gcopy(j + 1).start()
            wcopy(j).start()
        wcopy(n_chunks - 1).wait()

    return gather_kernel(ids_flat, table)


def kernel(input_ids, table):
    batch, seq = input_ids.shape
    total = batch * seq
    ids_flat = input_ids.reshape(total).astype(jnp.int32)
    assert total % (NUM_WORKERS * CHUNK) == 0
    n_chunks = total // (NUM_WORKERS * CHUNK)
    out = _sc_gather(ids_flat, table, n_chunks)
    return out.reshape(batch, seq, EMBED_DIM)


# speedup vs baseline: 1.0081x; 1.0029x over previous
import functools

import jax
import jax.numpy as jnp
from jax import lax
from jax.experimental import pallas as pl
from jax.experimental.pallas import tpu as pltpu
from jax.experimental.pallas import tpu_sc as plsc

EMBED_DIM = 64
NUM_CORES = 2
NUM_SUBCORES = 16
NUM_WORKERS = NUM_CORES * NUM_SUBCORES
CHUNK = 640
SUBSTREAMS = 8


@functools.partial(jax.jit, static_argnames=("n_chunks",))
def _sc_gather(ids_flat, table, n_chunks):
    bpw = ids_flat.shape[0] // NUM_WORKERS
    out_type = jax.ShapeDtypeStruct((ids_flat.shape[0], EMBED_DIM), table.dtype)
    mesh = plsc.VectorSubcoreMesh(core_axis_name="c", subcore_axis_name="s")

    sub = CHUNK // SUBSTREAMS

    @functools.partial(
        pl.kernel,
        mesh=mesh,
        out_type=out_type,
        scratch_types=[
            pltpu.VMEM((bpw,), jnp.int32),
            pltpu.VMEM((2, CHUNK, EMBED_DIM), jnp.float32),
            pltpu.SemaphoreType.DMA((2, SUBSTREAMS)),
            pltpu.SemaphoreType.DMA((2,)),
        ],
        compiler_params=pltpu.CompilerParams(use_tc_tiling_on_sc=False),
    )
    def gather_kernel(ids_hbm, table_hbm, out_hbm, idx_v, rows_v, sg, sw):
        wid = lax.axis_index("s") * NUM_CORES + lax.axis_index("c")
        base = wid * bpw

        pltpu.sync_copy(ids_hbm.at[pl.ds(base, bpw)], idx_v)

        def wcopy(t, p):
            return pltpu.make_async_copy(
                rows_v.at[p], out_hbm.at[pl.ds(base + t * CHUNK, CHUNK)],
                sw.at[p])

        def body(t, carry):
            p = lax.rem(t, 2)
            @pl.when(t >= 2)
            def _():
                wcopy(t - 2, p).wait()

            copies = []
            for s in range(SUBSTREAMS):
                c = pltpu.make_async_copy(
                    table_hbm.at[idx_v.at[pl.ds(t * CHUNK + s * sub, sub)]],
                    rows_v.at[p, pl.ds(s * sub, sub)],
                    sg.at[p, s])
                c.start()
                copies.append(c)
            for c in copies:
                c.wait()
            wcopy(t, p).start()
            return carry

        lax.fori_loop(0, n_chunks, body, 0)
        wcopy(n_chunks - 2, 0).wait()
        wcopy(n_chunks - 1, 1).wait()

    return gather_kernel(ids_flat, table)


def kernel(input_ids, table):
    batch, seq = input_ids.shape
    total = batch * seq
    ids_flat = input_ids.reshape(total).astype(jnp.int32)
    assert total % (NUM_WORKERS * CHUNK) == 0
    n_chunks = total // (NUM_WORKERS * CHUNK)
    assert n_chunks >= 2 and n_chunks % 2 == 0
    out = _sc_gather(ids_flat, table, n_chunks)
    return out.reshape(batch, seq, EMBED_DIM)
